# Initial kernel scaffold; baseline (speedup 1.0000x reference)
import jax
import jax.numpy as jnp
from jax.experimental import pallas as pl


def kernel(node_tokens, edge_tokens, edge_index, emb_table, W_gnn, W_edge, W_t, b_t, W_lm, b_lm):
    raise NotImplementedError("write your pallas kernel here")


# speedup vs baseline: 1.1342x; 1.1342x over previous
import jax
import jax.numpy as jnp
from jax.experimental import pallas as pl
from jax.experimental.pallas import tpu as pltpu

HID = 64


def _mm_body(x_ref, w_ref, o_ref):
    o_ref[...] = jnp.dot(x_ref[...], w_ref[...],
                         preferred_element_type=jnp.float32)


def _matmul(x, w, block_rows):
    n, k = x.shape
    m = w.shape[1]
    return pl.pallas_call(
        _mm_body,
        grid=(n // block_rows,),
        in_specs=[pl.BlockSpec((block_rows, k), lambda i: (i, 0)),
                  pl.BlockSpec((k, m), lambda i: (0, 0))],
        out_specs=pl.BlockSpec((block_rows, m), lambda i: (i, 0)),
        out_shape=jax.ShapeDtypeStruct((n, m), jnp.float32),
    )(x, w)


def _h2u_body(h_ref, agg_ref, wt_ref, u_ref):
    h2 = jnp.maximum(h_ref[...] + agg_ref[...], 0.0)
    u_ref[...] = jnp.dot(h2, wt_ref[...], preferred_element_type=jnp.float32)


def _tail_body(z_ref, bt_ref, wlm_ref, blm_ref, et_ref, lab_ref, probs_ref):
    et = et_ref[...]
    special = et <= 3
    masked = ((et * 131071) % 100 < 15) & (~special)
    lab_ref[...] = jnp.where(masked, et, -100)
    s = jnp.tanh(z_ref[...] + bt_ref[...][None, :])
    logits = jnp.dot(s, wlm_ref[...], preferred_element_type=jnp.float32)
    logits = logits + blm_ref[...][None, :]
    m = jnp.max(logits, axis=-1, keepdims=True)
    p = jnp.exp(logits - m)
    p = p / jnp.sum(p, axis=-1, keepdims=True)
    probs_ref[...] = p[:, None, :]


def kernel(node_tokens, edge_tokens, edge_index, emb_table, W_gnn, W_edge,
           W_t, b_t, W_lm, b_lm):
    n_nodes, l_node = node_tokens.shape
    n_edges, l_edge = edge_tokens.shape
    vocab = W_lm.shape[1]
    src, dst = edge_index[0], edge_index[1]

    node_emb = jnp.take(emb_table, node_tokens.reshape(-1), axis=0,
                        ).reshape(n_nodes, l_node * HID)
    edge_emb = jnp.take(emb_table, edge_tokens.reshape(-1), axis=0,
                        ).reshape(n_edges, l_edge * HID)

    h = _matmul(node_emb, W_gnn, 200)
    msg_e = _matmul(edge_emb, W_edge, 400)

    msg = h[src] + msg_e
    agg = jnp.zeros_like(h).at[dst].add(msg)

    u = pl.pallas_call(
        _h2u_body,
        grid=(1,),
        in_specs=[pl.BlockSpec((n_nodes, HID), lambda i: (0, 0)),
                  pl.BlockSpec((n_nodes, HID), lambda i: (0, 0)),
                  pl.BlockSpec((HID, HID), lambda i: (0, 0))],
        out_specs=pl.BlockSpec((n_nodes, HID), lambda i: (0, 0)),
        out_shape=jax.ShapeDtypeStruct((n_nodes, HID), jnp.float32),
    )(h, agg, W_t)

    z = u[src] + u[dst]

    rb = 200
    labels, probs = pl.pallas_call(
        _tail_body,
        grid=(n_edges // rb,),
        in_specs=[pl.BlockSpec((rb, HID), lambda i: (i, 0)),
                  pl.BlockSpec((HID,), lambda i: (0,)),
                  pl.BlockSpec((HID, vocab), lambda i: (0, 0)),
                  pl.BlockSpec((vocab,), lambda i: (0,)),
                  pl.BlockSpec((rb, l_edge), lambda i: (i, 0))],
        out_specs=[pl.BlockSpec((rb, l_edge), lambda i: (i, 0)),
                   pl.BlockSpec((rb, 1, vocab), lambda i: (i, 0, 0))],
        out_shape=[jax.ShapeDtypeStruct((n_edges, l_edge), jnp.int32),
                   jax.ShapeDtypeStruct((n_edges, 1, vocab), jnp.float32)],
    )(z, b_t, W_lm, b_lm, edge_tokens)

    return (labels, probs)


# speedup vs baseline: 2.2876x; 2.0168x over previous
import functools

import jax
import jax.numpy as jnp
from jax import lax
from jax.experimental import pallas as pl
from jax.experimental.pallas import tpu as pltpu
from jax.experimental.pallas import tpu_sc as plsc

HID = 64
_NW = 32
_CHUNK = 128
_NSLOT = 8


def _sc_gather(table, idx3):
    nw, nch, ck = idx3.shape
    d = table.shape[1]
    mesh = plsc.VectorSubcoreMesh(core_axis_name="c", subcore_axis_name="s")

    @functools.partial(
        pl.kernel, mesh=mesh,
        out_type=jax.ShapeDtypeStruct((nw * nch * ck, d), table.dtype),
        compiler_params=pltpu.CompilerParams(use_tc_tiling_on_sc=False),
        scratch_types=[
            pltpu.VMEM((nch, ck), jnp.int32),
            pltpu.VMEM((_NSLOT, ck, d), table.dtype),
            pltpu.SemaphoreType.DMA((_NSLOT,)),
            pltpu.SemaphoreType.DMA((_NSLOT,)),
        ])
    def gather_kernel(table_hbm, idx_hbm, out_hbm, idx_v, buf, gsem, ssem):
        wid = lax.axis_index("s") * 2 + lax.axis_index("c")
        base = wid * (nch * ck)
        pltpu.sync_copy(idx_hbm.at[wid], idx_v)
        for b in range(_NSLOT):
            pltpu.async_copy(table_hbm.at[idx_v.at[b]], buf.at[b], gsem.at[b])

        def drain_and_store(g, b):
            pltpu.make_async_copy(
                table_hbm.at[idx_v.at[0]], buf.at[b], gsem.at[b]).wait()
            pltpu.async_copy(
                buf.at[b], out_hbm.at[pl.ds(base + g * ck, ck)], ssem.at[b])

        @pl.loop(0, nch - _NSLOT, step=_NSLOT)
        def _(g0):
            for b in range(_NSLOT):
                g = g0 + b
                drain_and_store(g, b)
                pltpu.make_async_copy(
                    buf.at[b], out_hbm.at[pl.ds(base, ck)], ssem.at[b]).wait()
                pltpu.async_copy(
                    table_hbm.at[idx_v.at[g + _NSLOT]], buf.at[b], gsem.at[b])

        for b in range(_NSLOT):
            drain_and_store(nch - _NSLOT + b, b)
        for b in range(_NSLOT):
            pltpu.make_async_copy(
                buf.at[b], out_hbm.at[pl.ds(base, ck)], ssem.at[b]).wait()

    return gather_kernel(table, idx3)


def _mm_body(x_ref, w_ref, o_ref):
    o_ref[...] = jnp.dot(x_ref[...], w_ref[...],
                         preferred_element_type=jnp.float32)


def _matmul(x, w, block_rows):
    n, k = x.shape
    m = w.shape[1]
    return pl.pallas_call(
        _mm_body,
        grid=(n // block_rows,),
        in_specs=[pl.BlockSpec((block_rows, k), lambda i: (i, 0)),
                  pl.BlockSpec((k, m), lambda i: (0, 0))],
        out_specs=pl.BlockSpec((block_rows, m), lambda i: (i, 0)),
        out_shape=jax.ShapeDtypeStruct((n, m), jnp.float32),
    )(x, w)


def _h2u_body(h_ref, agg_ref, wt_ref, u_ref):
    h2 = jnp.maximum(h_ref[...] + agg_ref[...], 0.0)
    u_ref[...] = jnp.dot(h2, wt_ref[...], preferred_element_type=jnp.float32)


def _tail_body(z_ref, bt_ref, wlm_ref, blm_ref, et_ref, lab_ref, probs_ref):
    et = et_ref[...]
    special = et <= 3
    masked = ((et * 131071) % 100 < 15) & (~special)
    lab_ref[...] = jnp.where(masked, et, -100)
    s = jnp.tanh(z_ref[...] + bt_ref[...][None, :])
    logits = jnp.dot(s, wlm_ref[...], preferred_element_type=jnp.float32)
    logits = logits + blm_ref[...][None, :]
    m = jnp.max(logits, axis=-1, keepdims=True)
    p = jnp.exp(logits - m)
    p = p / jnp.sum(p, axis=-1, keepdims=True)
    probs_ref[...] = p[:, None, :]


def kernel(node_tokens, edge_tokens, edge_index, emb_table, W_gnn, W_edge,
           W_t, b_t, W_lm, b_lm):
    n_nodes, l_node = node_tokens.shape
    n_edges, l_edge = edge_tokens.shape
    vocab = W_lm.shape[1]
    src, dst = edge_index[0], edge_index[1]

    n_flat = node_tokens.reshape(-1)
    e_flat = edge_tokens.reshape(-1)
    tot = n_flat.size + e_flat.size
    nch = -(-tot // (_NW * _CHUNK))
    nch = -(-nch // _NSLOT) * _NSLOT
    pad = _NW * nch * _CHUNK - tot
    flat = jnp.concatenate(
        [n_flat, e_flat, jnp.zeros((pad,), n_flat.dtype)])
    rows = _sc_gather(emb_table, flat.reshape(_NW, nch, _CHUNK))
    node_emb = rows[:n_flat.size].reshape(n_nodes, l_node * HID)
    edge_emb = rows[n_flat.size:tot].reshape(n_edges, l_edge * HID)

    h = _matmul(node_emb, W_gnn, 200)
    msg_e = _matmul(edge_emb, W_edge, 400)

    msg = h[src] + msg_e
    agg = jnp.zeros_like(h).at[dst].add(msg)

    u = pl.pallas_call(
        _h2u_body,
        grid=(1,),
        in_specs=[pl.BlockSpec((n_nodes, HID), lambda i: (0, 0)),
                  pl.BlockSpec((n_nodes, HID), lambda i: (0, 0)),
                  pl.BlockSpec((HID, HID), lambda i: (0, 0))],
        out_specs=pl.BlockSpec((n_nodes, HID), lambda i: (0, 0)),
        out_shape=jax.ShapeDtypeStruct((n_nodes, HID), jnp.float32),
    )(h, agg, W_t)

    z = u[src] + u[dst]

    rb = 200
    labels, probs = pl.pallas_call(
        _tail_body,
        grid=(n_edges // rb,),
        in_specs=[pl.BlockSpec((rb, HID), lambda i: (i, 0)),
                  pl.BlockSpec((HID,), lambda i: (0,)),
                  pl.BlockSpec((HID, vocab), lambda i: (0, 0)),
                  pl.BlockSpec((vocab,), lambda i: (0,)),
                  pl.BlockSpec((rb, l_edge), lambda i: (i, 0))],
        out_specs=[pl.BlockSpec((rb, l_edge), lambda i: (i, 0)),
                   pl.BlockSpec((rb, 1, vocab), lambda i: (i, 0, 0))],
        out_shape=[jax.ShapeDtypeStruct((n_edges, l_edge), jnp.int32),
                   jax.ShapeDtypeStruct((n_edges, 1, vocab), jnp.float32)],
    )(z, b_t, W_lm, b_lm, edge_tokens)

    return (labels, probs)


# speedup vs baseline: 2.7001x; 1.1803x over previous
import functools

import jax
import jax.numpy as jnp
from jax import lax
from jax.experimental import pallas as pl
from jax.experimental.pallas import tpu as pltpu
from jax.experimental.pallas import tpu_sc as plsc

HID = 64
_NW = 32
_CHUNK = 128
_NSLOT = 8


def _sc_gather(table, idx3):
    nw, nch, ck = idx3.shape
    d = table.shape[1]
    mesh = plsc.VectorSubcoreMesh(core_axis_name="c", subcore_axis_name="s")

    @functools.partial(
        pl.kernel, mesh=mesh,
        out_type=jax.ShapeDtypeStruct((nw * nch * ck, d), table.dtype),
        compiler_params=pltpu.CompilerParams(use_tc_tiling_on_sc=False),
        scratch_types=[
            pltpu.VMEM((nch, ck), jnp.int32),
            pltpu.VMEM((_NSLOT, ck, d), table.dtype),
            pltpu.SemaphoreType.DMA((_NSLOT,)),
            pltpu.SemaphoreType.DMA((_NSLOT,)),
        ])
    def gather_kernel(table_hbm, idx_hbm, out_hbm, idx_v, buf, gsem, ssem):
        wid = lax.axis_index("s") * 2 + lax.axis_index("c")
        base = wid * (nch * ck)
        pltpu.sync_copy(idx_hbm.at[wid], idx_v)
        for b in range(_NSLOT):
            pltpu.async_copy(table_hbm.at[idx_v.at[b]], buf.at[b], gsem.at[b])

        def drain_and_store(g, b):
            pltpu.make_async_copy(
                table_hbm.at[idx_v.at[0]], buf.at[b], gsem.at[b]).wait()
            pltpu.async_copy(
                buf.at[b], out_hbm.at[pl.ds(base + g * ck, ck)], ssem.at[b])

        @pl.loop(0, nch - _NSLOT, step=_NSLOT)
        def _(g0):
            for b in range(_NSLOT):
                g = g0 + b
                drain_and_store(g, b)
                pltpu.make_async_copy(
                    buf.at[b], out_hbm.at[pl.ds(base, ck)], ssem.at[b]).wait()
                pltpu.async_copy(
                    table_hbm.at[idx_v.at[g + _NSLOT]], buf.at[b], gsem.at[b])

        for b in range(_NSLOT):
            drain_and_store(nch - _NSLOT + b, b)
        for b in range(_NSLOT):
            pltpu.make_async_copy(
                buf.at[b], out_hbm.at[pl.ds(base, ck)], ssem.at[b]).wait()

    return gather_kernel(table, idx3)


def _mm_body(x_ref, w_ref, o_ref):
    o_ref[...] = jnp.dot(x_ref[...], w_ref[...].astype(x_ref.dtype),
                         preferred_element_type=jnp.float32)


def _matmul(x, w, block_rows):
    n, k = x.shape
    m = w.shape[1]
    return pl.pallas_call(
        _mm_body,
        grid=(n // block_rows,),
        in_specs=[pl.BlockSpec((block_rows, k), lambda i: (i, 0)),
                  pl.BlockSpec((k, m), lambda i: (0, 0))],
        out_specs=pl.BlockSpec((block_rows, m), lambda i: (i, 0)),
        out_shape=jax.ShapeDtypeStruct((n, m), jnp.float32),
    )(x, w)


def _h2u_body(h_ref, agg_ref, wt_ref, u_ref):
    h2 = jnp.maximum(h_ref[...] + agg_ref[...], 0.0)
    u_ref[...] = jnp.dot(h2, wt_ref[...], preferred_element_type=jnp.float32)


def _tail_body(z_ref, bt_ref, wlm_ref, blm_ref, et_ref, lab_ref, probs_ref):
    et = et_ref[...]
    special = et <= 3
    masked = ((et * 131071) % 100 < 15) & (~special)
    lab_ref[...] = jnp.where(masked, et, -100)
    s = jnp.tanh(z_ref[...] + bt_ref[...][None, :])
    logits = jnp.dot(s, wlm_ref[...], preferred_element_type=jnp.float32)
    logits = logits + blm_ref[...][None, :]
    m = jnp.max(logits, axis=-1, keepdims=True)
    p = jnp.exp(logits - m)
    p = p / jnp.sum(p, axis=-1, keepdims=True)
    probs_ref[...] = p[:, None, :]


def kernel(node_tokens, edge_tokens, edge_index, emb_table, W_gnn, W_edge,
           W_t, b_t, W_lm, b_lm):
    n_nodes, l_node = node_tokens.shape
    n_edges, l_edge = edge_tokens.shape
    vocab = W_lm.shape[1]
    src, dst = edge_index[0], edge_index[1]

    n_flat = node_tokens.reshape(-1)
    e_flat = edge_tokens.reshape(-1)
    tot = n_flat.size + e_flat.size
    nch = -(-tot // (_NW * _CHUNK))
    nch = -(-nch // _NSLOT) * _NSLOT
    pad = _NW * nch * _CHUNK - tot
    flat = jnp.concatenate(
        [n_flat, e_flat, jnp.zeros((pad,), n_flat.dtype)])
    rows = _sc_gather(emb_table.astype(jnp.bfloat16),
                      flat.reshape(_NW, nch, _CHUNK))
    node_emb = rows[:n_flat.size].reshape(n_nodes, l_node * HID)
    edge_emb = rows[n_flat.size:tot].reshape(n_edges, l_edge * HID)

    h = _matmul(node_emb, W_gnn, 200)
    msg_e = _matmul(edge_emb, W_edge, 400)

    msg = h[src] + msg_e
    agg = jnp.zeros_like(h).at[dst].add(msg)

    u = pl.pallas_call(
        _h2u_body,
        grid=(1,),
        in_specs=[pl.BlockSpec((n_nodes, HID), lambda i: (0, 0)),
                  pl.BlockSpec((n_nodes, HID), lambda i: (0, 0)),
                  pl.BlockSpec((HID, HID), lambda i: (0, 0))],
        out_specs=pl.BlockSpec((n_nodes, HID), lambda i: (0, 0)),
        out_shape=jax.ShapeDtypeStruct((n_nodes, HID), jnp.float32),
    )(h, agg, W_t)

    z = u[src] + u[dst]

    rb = 200
    labels, probs = pl.pallas_call(
        _tail_body,
        grid=(n_edges // rb,),
        in_specs=[pl.BlockSpec((rb, HID), lambda i: (i, 0)),
                  pl.BlockSpec((HID,), lambda i: (0,)),
                  pl.BlockSpec((HID, vocab), lambda i: (0, 0)),
                  pl.BlockSpec((vocab,), lambda i: (0,)),
                  pl.BlockSpec((rb, l_edge), lambda i: (i, 0))],
        out_specs=[pl.BlockSpec((rb, l_edge), lambda i: (i, 0)),
                   pl.BlockSpec((rb, 1, vocab), lambda i: (i, 0, 0))],
        out_shape=[jax.ShapeDtypeStruct((n_edges, l_edge), jnp.int32),
                   jax.ShapeDtypeStruct((n_edges, 1, vocab), jnp.float32)],
    )(z, b_t, W_lm, b_lm, edge_tokens)

    return (labels, probs)


# speedup vs baseline: 4.5610x; 1.6892x over previous
import functools

import jax
import jax.numpy as jnp
from jax import lax
from jax.experimental import pallas as pl
from jax.experimental.pallas import tpu as pltpu
from jax.experimental.pallas import tpu_sc as plsc

HID = 64
_NW = 32
_CHUNK = 128
_NSLOT = 8


def _sc_gather(table32, idx3, n_node_ch, n_edge_ch):
    nw, nch, ck = idx3.shape
    d = table32.shape[1]
    rpc = ck * d // 128
    n_dump_ch = nw * nch - n_node_ch - n_edge_ch
    mesh = plsc.VectorSubcoreMesh(core_axis_name="c", subcore_axis_name="s")

    @functools.partial(
        pl.kernel, mesh=mesh,
        out_type=[
            jax.ShapeDtypeStruct((n_node_ch * ck, d), jnp.float32),
            jax.ShapeDtypeStruct((n_edge_ch * ck, d), jnp.float32),
            jax.ShapeDtypeStruct((n_dump_ch * ck, d), jnp.float32),
        ],
        compiler_params=pltpu.CompilerParams(use_tc_tiling_on_sc=False),
        scratch_types=[
            pltpu.VMEM((nch, ck), jnp.int32),
            pltpu.VMEM((_NSLOT, ck, d), jnp.float32),
            pltpu.SemaphoreType.DMA((_NSLOT,)),
            pltpu.SemaphoreType.DMA((_NSLOT,)),
        ])
    def gather_kernel(table_hbm, idx_hbm, node_hbm, edge_hbm, dump_hbm,
                      idx_v, buf, gsem, ssem):
        wid = lax.axis_index("s") * 2 + lax.axis_index("c")
        gbase = wid * nch
        cksz = ck * d
        pltpu.sync_copy(idx_hbm.at[wid], idx_v)
        for b in range(_NSLOT):
            pltpu.async_copy(table_hbm.at[idx_v.at[b]], buf.at[b], gsem.at[b])

        def store_chunk(g, b):
            pltpu.make_async_copy(
                table_hbm.at[idx_v.at[0]], buf.at[b], gsem.at[b]).wait()
            gg = gbase + g
            src = buf.at[b]

            @pl.when(gg < n_node_ch)
            def _():
                pltpu.async_copy(
                    src, node_hbm.at[pl.ds(gg * ck, ck)], ssem.at[b])

            @pl.when(jnp.logical_and(gg >= n_node_ch,
                                     gg < n_node_ch + n_edge_ch))
            def _():
                pltpu.async_copy(
                    src, edge_hbm.at[pl.ds((gg - n_node_ch) * ck, ck)],
                    ssem.at[b])

            @pl.when(gg >= n_node_ch + n_edge_ch)
            def _():
                pltpu.async_copy(
                    src,
                    dump_hbm.at[pl.ds((gg - n_node_ch - n_edge_ch) * ck, ck)],
                    ssem.at[b])

        def wait_store(b):
            pltpu.make_async_copy(
                buf.at[b], node_hbm.at[pl.ds(0, ck)], ssem.at[b]).wait()

        @pl.loop(0, nch - _NSLOT, step=_NSLOT)
        def _(g0):
            for b in range(_NSLOT):
                g = g0 + b
                store_chunk(g, b)
                wait_store(b)
                pltpu.async_copy(
                    table_hbm.at[idx_v.at[g + _NSLOT]], buf.at[b],
                    gsem.at[b])

        for b in range(_NSLOT):
            store_chunk(nch - _NSLOT + b, b)
        for b in range(_NSLOT):
            wait_store(b)

    return gather_kernel(table32, idx3)


def _acc_mm_body(x_ref, we_ref, wo_ref, o_ref):
    g = pl.program_id(0)
    x = x_ref[...]
    m = x.shape[0]
    xb = pltpu.bitcast(x, jnp.bfloat16)
    xr = xb.reshape(m, 2, 128)
    x2 = jnp.concatenate([xr[:, 0, :], xr[:, 1, :]], axis=1)
    w2 = jnp.concatenate([we_ref[0], wo_ref[0]], axis=0)
    acc = jnp.dot(x2, w2, preferred_element_type=jnp.float32)

    @pl.when(g == 0)
    def _():
        o_ref[...] = jnp.zeros_like(o_ref)

    o_ref[...] += acc


def _grouped_matmul(rows, we3, wo3, n_rows):
    ng = we3.shape[0]
    return pl.pallas_call(
        _acc_mm_body,
        grid=(ng,),
        in_specs=[pl.BlockSpec((n_rows, 128), lambda g: (g, 0)),
                  pl.BlockSpec((1, 128, HID), lambda g: (g, 0, 0)),
                  pl.BlockSpec((1, 128, HID), lambda g: (g, 0, 0))],
        out_specs=pl.BlockSpec((n_rows, HID), lambda g: (0, 0)),
        out_shape=jax.ShapeDtypeStruct((n_rows, HID), jnp.float32),
    )(rows, we3, wo3)


def _split_weight(w, seq_len):
    wr = w.reshape(seq_len // 4, 4, HID, HID)
    we = wr[:, :, 0::2, :].reshape(-1, 128, HID).astype(jnp.bfloat16)
    wo = wr[:, :, 1::2, :].reshape(-1, 128, HID).astype(jnp.bfloat16)
    return we, wo


def _h2u_body(h_ref, agg_ref, wt_ref, u_ref):
    h2 = jnp.maximum(h_ref[...] + agg_ref[...], 0.0)
    u_ref[...] = jnp.dot(h2, wt_ref[...], preferred_element_type=jnp.float32)


def _tail_body(z_ref, bt_ref, wlm_ref, blm_ref, et_ref, lab_ref, probs_ref):
    et = et_ref[...]
    special = et <= 3
    masked = ((et * 131071) % 100 < 15) & (~special)
    lab_ref[...] = jnp.where(masked, et, -100)
    s = jnp.tanh(z_ref[...] + bt_ref[...][None, :])
    logits = jnp.dot(s, wlm_ref[...], preferred_element_type=jnp.float32)
    logits = logits + blm_ref[...][None, :]
    m = jnp.max(logits, axis=-1, keepdims=True)
    p = jnp.exp(logits - m)
    p = p / jnp.sum(p, axis=-1, keepdims=True)
    probs_ref[...] = p[:, None, :]


def kernel(node_tokens, edge_tokens, edge_index, emb_table, W_gnn, W_edge,
           W_t, b_t, W_lm, b_lm):
    n_nodes, l_node = node_tokens.shape
    n_edges, l_edge = edge_tokens.shape
    vocab = W_lm.shape[1]
    src, dst = edge_index[0], edge_index[1]

    tab16 = emb_table.astype(jnp.bfloat16)
    tab32 = lax.bitcast_convert_type(
        tab16.reshape(emb_table.shape[0], HID // 2, 2), jnp.float32)

    idx_n = node_tokens.reshape(n_nodes, l_node // 4, 4
                                ).transpose(1, 0, 2).reshape(-1)
    idx_e = edge_tokens.reshape(n_edges, l_edge // 4, 4
                                ).transpose(1, 0, 2).reshape(-1)
    n_node_ch = idx_n.size // _CHUNK
    n_edge_ch = idx_e.size // _CHUNK
    tot = idx_n.size + idx_e.size
    nch = -(-tot // (_NW * _CHUNK))
    nch = -(-nch // _NSLOT) * _NSLOT
    pad = _NW * nch * _CHUNK - tot
    flat = jnp.concatenate([idx_n, idx_e, jnp.zeros((pad,), idx_n.dtype)])
    rows_n, rows_e, _ = _sc_gather(
        tab32, flat.reshape(_NW, nch, _CHUNK), n_node_ch, n_edge_ch)

    we_n, wo_n = _split_weight(W_gnn, l_node)
    we_e, wo_e = _split_weight(W_edge, l_edge)
    h = _grouped_matmul(rows_n.reshape(-1, 128), we_n, wo_n, n_nodes)
    msg_e = _grouped_matmul(rows_e.reshape(-1, 128), we_e, wo_e, n_edges)

    msg = h[src] + msg_e
    agg = jnp.zeros_like(h).at[dst].add(msg)

    u = pl.pallas_call(
        _h2u_body,
        grid=(1,),
        in_specs=[pl.BlockSpec((n_nodes, HID), lambda i: (0, 0)),
                  pl.BlockSpec((n_nodes, HID), lambda i: (0, 0)),
                  pl.BlockSpec((HID, HID), lambda i: (0, 0))],
        out_specs=pl.BlockSpec((n_nodes, HID), lambda i: (0, 0)),
        out_shape=jax.ShapeDtypeStruct((n_nodes, HID), jnp.float32),
    )(h, agg, W_t)

    z = u[src] + u[dst]

    rb = 200
    labels, probs = pl.pallas_call(
        _tail_body,
        grid=(n_edges // rb,),
        in_specs=[pl.BlockSpec((rb, HID), lambda i: (i, 0)),
                  pl.BlockSpec((HID,), lambda i: (0,)),
                  pl.BlockSpec((HID, vocab), lambda i: (0, 0)),
                  pl.BlockSpec((vocab,), lambda i: (0,)),
                  pl.BlockSpec((rb, l_edge), lambda i: (i, 0))],
        out_specs=[pl.BlockSpec((rb, l_edge), lambda i: (i, 0)),
                   pl.BlockSpec((rb, 1, vocab), lambda i: (i, 0, 0))],
        out_shape=[jax.ShapeDtypeStruct((n_edges, l_edge), jnp.int32),
                   jax.ShapeDtypeStruct((n_edges, 1, vocab), jnp.float32)],
    )(z, b_t, W_lm, b_lm, edge_tokens)

    return (labels, probs)
